# baseline (speedup 1.0000x reference)
import functools

import jax
import jax.numpy as jnp
from jax import lax
from jax.experimental import pallas as pl
from jax.experimental.pallas import tpu as pltpu
from jax.experimental.pallas import tpu_sc as plsc

VOCAB = 1_000_000
DIM = 64
BATCH = 4096
HIST = 200
N = BATCH * HIST

NC, NS = 2, 16
NW = NC * NS
ROWS_PER_W = N // NW
CHUNK = 128
CHUNKS_PER_W = ROWS_PER_W // CHUNK
NBUF = 4
NBANK = 2
GROUP = NBUF * CHUNK
NGROUP = CHUNKS_PER_W // NBUF
STEPS = NGROUP // NBANK


def _gather_body(idx_hbm, table_hbm, out_hbm, idx_v, rows_v, gsems, ssems):
    c = lax.axis_index("c")
    s = lax.axis_index("s")
    wid = s * NC + c
    base = wid * ROWS_PER_W

    pltpu.sync_copy(idx_hbm.at[wid], idx_v)

    def fire_gather(p, b, chunk):
        return pltpu.async_copy(
            table_hbm.at[idx_v.at[chunk]], rows_v.at[p, b], gsems.at[p, b])

    def fire_store(p, b, chunk):
        return pltpu.async_copy(
            rows_v.at[p, b],
            out_hbm.at[pl.ds(base + chunk * CHUNK, CHUNK)],
            ssems.at[p, b])

    def wait_gather(p, b, chunk):
        pltpu.make_async_copy(
            table_hbm.at[idx_v.at[chunk]], rows_v.at[p, b], gsems.at[p, b]
        ).wait()

    def wait_store(p, b, chunk):
        pltpu.make_async_copy(
            rows_v.at[p, b],
            out_hbm.at[pl.ds(base + chunk * CHUNK, CHUNK)],
            ssems.at[p, b]
        ).wait()

    for p in range(NBANK):
        for b in range(NBUF):
            fire_gather(p, b, p * NBUF + b)

    def step(g, _):
        for p in range(NBANK):
            grp = g * NBANK + p
            for b in range(NBUF):
                chunk = grp * NBUF + b
                wait_gather(p, b, chunk)
                fire_store(p, b, chunk)
            nxt = grp + NBANK
            for b in range(NBUF):
                chunk = grp * NBUF + b
                nxt_chunk = nxt * NBUF + b

                @pl.when(nxt < NGROUP)
                def _():
                    wait_store(p, b, chunk)
                    fire_gather(p, b, nxt_chunk)
        return _

    lax.fori_loop(0, STEPS, step, None)

    last = (NGROUP - NBANK) * NBUF
    for p in range(NBANK):
        for b in range(NBUF):
            wait_store(p, b, last + p * NBUF + b)


@functools.partial(jax.jit, static_argnames=())
def _sc_gather(idx, weight):
    mesh = plsc.VectorSubcoreMesh(
        core_axis_name="c", subcore_axis_name="s",
        num_cores=NC, num_subcores=NS)
    k = pl.kernel(
        _gather_body,
        out_type=jax.ShapeDtypeStruct((N, DIM), jnp.float32),
        mesh=mesh,
        compiler_params=pltpu.CompilerParams(use_tc_tiling_on_sc=False),
        scratch_types=[
            pltpu.VMEM((CHUNKS_PER_W, CHUNK), jnp.int32),
            pltpu.VMEM((NBANK, NBUF, CHUNK, DIM), jnp.float32),
            pltpu.SemaphoreType.DMA((NBANK, NBUF)),
            pltpu.SemaphoreType.DMA((NBANK, NBUF)),
        ],
    )
    return k(idx, weight)


def kernel(token_ids, weight):
    idx = token_ids.astype(jnp.int32).reshape(NW, CHUNKS_PER_W, CHUNK)
    out = _sc_gather(idx, weight)
    return out.reshape(BATCH, HIST, DIM), weight


# speedup vs baseline: 1.0031x; 1.0031x over previous
import functools

import jax
import jax.numpy as jnp
from jax import lax
from jax.experimental import pallas as pl
from jax.experimental.pallas import tpu as pltpu
from jax.experimental.pallas import tpu_sc as plsc

VOCAB = 1_000_000
DIM = 64
BATCH = 4096
HIST = 200
N = BATCH * HIST

NC, NS = 2, 16
NW = NC * NS
ROWS_PER_W = N // NW
CHUNK = 128
NBUF = 4
GROUP = NBUF * CHUNK
NGROUP = ROWS_PER_W // GROUP
NBANK = 2
STEPS = NGROUP - NBANK


def _gather_body(idx_hbm, table_hbm, out_hbm, idx_v, rows_v, gsems, ssems):
    c = lax.axis_index("c")
    s = lax.axis_index("s")
    wid = s * NC + c
    gbase = wid * NGROUP

    pltpu.sync_copy(idx_hbm.at[wid], idx_v)

    def fire_gather(p, grp):
        return pltpu.async_copy(
            table_hbm.at[idx_v.at[grp]], rows_v.at[p], gsems.at[p])

    def fire_store(p, grp):
        return pltpu.async_copy(
            rows_v.at[p], out_hbm.at[gbase + grp], ssems.at[p])

    def wait_gather(p, grp):
        pltpu.make_async_copy(
            table_hbm.at[idx_v.at[grp]], rows_v.at[p], gsems.at[p]).wait()

    def wait_store(p, grp):
        pltpu.make_async_copy(
            rows_v.at[p], out_hbm.at[gbase + grp], ssems.at[p]).wait()

    for p in range(NBANK):
        fire_gather(p, p)

    def step(g, _):
        for p in range(NBANK):
            grp = g * NBANK + p
            wait_gather(p, grp)
            fire_store(p, grp)
            wait_store(p, grp)
            fire_gather(p, grp + NBANK)
        return _

    lax.fori_loop(0, STEPS // NBANK, step, None)

    for p in range(NBANK):
        grp = STEPS + p
        wait_gather(p, grp)
        fire_store(p, grp)
    for p in range(NBANK):
        wait_store(p, STEPS + p)


@jax.jit
def _sc_gather(idx, weight):
    mesh = plsc.VectorSubcoreMesh(
        core_axis_name="c", subcore_axis_name="s",
        num_cores=NC, num_subcores=NS)
    k = pl.kernel(
        _gather_body,
        out_type=jax.ShapeDtypeStruct((N // GROUP, GROUP, DIM), jnp.float32),
        mesh=mesh,
        compiler_params=pltpu.CompilerParams(use_tc_tiling_on_sc=False),
        scratch_types=[
            pltpu.VMEM((NGROUP, GROUP), jnp.int32),
            pltpu.VMEM((NBANK, GROUP, DIM), jnp.float32),
            pltpu.SemaphoreType.DMA((NBANK,)),
            pltpu.SemaphoreType.DMA((NBANK,)),
        ],
    )
    return k(idx, weight)


def kernel(token_ids, weight):
    idx = token_ids.astype(jnp.int32).reshape(NW, NGROUP, GROUP)
    out = _sc_gather(idx, weight)
    return out.reshape(BATCH, HIST, DIM), weight
